# Initial kernel scaffold; baseline (speedup 1.0000x reference)
import functools

import jax
import jax.numpy as jnp
from jax import lax
from jax.experimental import pallas as pl
from jax.experimental.pallas import tpu as pltpu
from jax.experimental.pallas import tpu_sc as plsc

N_NODES_P = 10240
E_PER_TILE = 10240
N_CHUNKS = 80
STAGE_CHUNKS = 8
N_STAGES = N_CHUNKS // STAGE_CHUNKS
CHUNK = 128
NC = 2
NS = 16
E_PAD = NC * NS * E_PER_TILE
ROWS_PER_TILE = N_NODES_P // NS


def _sc_scatter_body(ea_hbm, src_hbm, sums_out, cnts_out,
                     idx_buf, ea_buf, ones_buf, zb, acc, cnt):
    c = lax.axis_index("c")
    s = lax.axis_index("s")
    w = s * NC + c

    def fill(i, _):
        ones_buf[i, :] = jnp.full((16,), 1.0, dtype=jnp.float32)
        return 0
    lax.fori_loop(0, CHUNK, fill, 0)

    def zfill(i, _):
        zb[i, :] = jnp.zeros((16,), dtype=jnp.float32)
        return 0
    lax.fori_loop(0, ROWS_PER_TILE, zfill, 0)

    off = s * ROWS_PER_TILE
    pltpu.sync_copy(zb, acc.at[pl.ds(off, ROWS_PER_TILE)])
    pltpu.sync_copy(zb, cnt.at[pl.ds(off, ROWS_PER_TILE)])
    plsc.subcore_barrier()

    pltpu.sync_copy(src_hbm.at[w], idx_buf)

    base = w * E_PER_TILE

    def stage_body(st, _):
        pltpu.sync_copy(
            ea_hbm.at[pl.ds(base + st * (STAGE_CHUNKS * CHUNK),
                            STAGE_CHUNKS * CHUNK)], ea_buf)
        for j in range(STAGE_CHUNKS):
            k = st * STAGE_CHUNKS + j
            pltpu.sync_copy(ea_buf.at[pl.ds(j * CHUNK, CHUNK)],
                            acc.at[idx_buf.at[k]], add=True)
            pltpu.sync_copy(ones_buf, cnt.at[idx_buf.at[k]], add=True)
        return 0
    lax.fori_loop(0, N_STAGES, stage_body, 0)

    plsc.subcore_barrier()

    pltpu.sync_copy(acc.at[pl.ds(off, ROWS_PER_TILE)], zb)
    pltpu.sync_copy(zb, sums_out.at[c, pl.ds(off, ROWS_PER_TILE)])
    pltpu.sync_copy(cnt.at[pl.ds(off, ROWS_PER_TILE)], zb)
    pltpu.sync_copy(zb, cnts_out.at[c, pl.ds(off, ROWS_PER_TILE)])


def _sc_scatter(ea_pad, src_pad):
    mesh = plsc.VectorSubcoreMesh(core_axis_name="c", subcore_axis_name="s")
    fn = pl.kernel(
        _sc_scatter_body,
        out_type=(
            jax.ShapeDtypeStruct((NC, N_NODES_P, 16), jnp.float32),
            jax.ShapeDtypeStruct((NC, N_NODES_P, 16), jnp.float32),
        ),
        mesh=mesh,
        scratch_types=[
            pltpu.VMEM((N_CHUNKS, CHUNK), jnp.int32),
            pltpu.VMEM((STAGE_CHUNKS * CHUNK, 16), jnp.float32),
            pltpu.VMEM((CHUNK, 16), jnp.float32),
            pltpu.VMEM((ROWS_PER_TILE, 16), jnp.float32),
            pltpu.VMEM_SHARED((N_NODES_P, 16), jnp.float32),
            pltpu.VMEM_SHARED((N_NODES_P, 16), jnp.float32),
        ],
    )
    return fn(ea_pad, src_pad)


def _mlp_body(x_ref, s_ref, c_ref, b_ref, u_ref, w1x_ref, w1e_ref, w1u_ref,
              b1_ref, w2_ref, b2_ref, o_ref):
    sums = s_ref[0] + s_ref[1]
    counts = c_ref[0, :, 0:1] + c_ref[1, :, 0:1]
    agg = sums / jnp.maximum(counts, 1.0)

    bvec = b_ref[...]
    gids = lax.broadcasted_iota(jnp.int32, (bvec.shape[0], 64), 1)
    onehot = (bvec == gids).astype(jnp.float32)

    uw = jnp.dot(u_ref[...], w1u_ref[...], preferred_element_type=jnp.float32)
    pre = (jnp.dot(x_ref[...], w1x_ref[...], preferred_element_type=jnp.float32)
           + jnp.dot(agg, w1e_ref[...], preferred_element_type=jnp.float32)
           + jnp.dot(onehot, uw, preferred_element_type=jnp.float32)
           + b1_ref[...])
    h = jnp.maximum(pre, 0.0)
    o_ref[...] = jnp.dot(h, w2_ref[...], preferred_element_type=jnp.float32) + b2_ref[...]


def _mlp(x, sums, cnts, batch2d, u, w1x, w1e, w1u, b1, w2, b2):
    n = x.shape[0]
    blk = 1000
    grid = n // blk
    return pl.pallas_call(
        _mlp_body,
        grid=(grid,),
        in_specs=[
            pl.BlockSpec((blk, 128), lambda i: (i, 0)),
            pl.BlockSpec((NC, blk, 16), lambda i: (0, i, 0)),
            pl.BlockSpec((NC, blk, 16), lambda i: (0, i, 0)),
            pl.BlockSpec((blk, 1), lambda i: (i, 0)),
            pl.BlockSpec((64, 128), lambda i: (0, 0)),
            pl.BlockSpec((128, 128), lambda i: (0, 0)),
            pl.BlockSpec((16, 128), lambda i: (0, 0)),
            pl.BlockSpec((128, 128), lambda i: (0, 0)),
            pl.BlockSpec((1, 128), lambda i: (0, 0)),
            pl.BlockSpec((128, 128), lambda i: (0, 0)),
            pl.BlockSpec((1, 128), lambda i: (0, 0)),
        ],
        out_specs=pl.BlockSpec((blk, 128), lambda i: (i, 0)),
        out_shape=jax.ShapeDtypeStruct((n, 128), jnp.float32),
    )(x, sums, cnts, batch2d, u, w1x, w1e, w1u, b1, w2, b2)


@jax.jit
def kernel(x, edge_index, edge_attr, u, batch, W1, b1, W2, b2):
    n_edges = edge_attr.shape[0]
    src = edge_index[0].astype(jnp.int32)
    src_pad = jnp.pad(src, (0, E_PAD - n_edges), constant_values=N_NODES_P - 1)
    src_pad = src_pad.reshape(NC * NS, N_CHUNKS, CHUNK)
    ea_pad = jnp.pad(edge_attr, ((0, E_PAD - n_edges), (0, 0)))

    sums, cnts = _sc_scatter(ea_pad, src_pad)

    w1x = W1[:128]
    w1e = W1[128:144]
    w1u = W1[144:]
    batch2d = batch.astype(jnp.int32).reshape(-1, 1)
    out = _mlp(x, sums[:, :x.shape[0]], cnts[:, :x.shape[0]], batch2d, u,
               w1x, w1e, w1u, b1.reshape(1, -1), W2, b2.reshape(1, -1))
    return out


# speedup vs baseline: 4.2156x; 4.2156x over previous
import functools

import jax
import jax.numpy as jnp
from jax import lax
from jax.experimental import pallas as pl
from jax.experimental.pallas import tpu as pltpu
from jax.experimental.pallas import tpu_sc as plsc

N_NODES_P = 10240
E_PER_TILE = 10240
N_CHUNKS = 80
STAGE_CHUNKS = 8
N_STAGES = N_CHUNKS // STAGE_CHUNKS
CHUNK = 128
NC = 2
NS = 16
E_PAD = NC * NS * E_PER_TILE
ROWS_PER_TILE = N_NODES_P // NS


def _sc_scatter_body(ea_hbm, src_hbm, sums_out, cnts_out,
                     idx_buf, ea_buf, ones_buf, zb, acc, cnt):
    c = lax.axis_index("c")
    s = lax.axis_index("s")
    w = s * NC + c

    def fill(i, _):
        ones_buf[i, :] = jnp.full((16,), 1.0, dtype=jnp.float32)
        return 0
    lax.fori_loop(0, CHUNK, fill, 0)

    def zfill(i, _):
        zb[i, :] = jnp.zeros((16,), dtype=jnp.float32)
        return 0
    lax.fori_loop(0, ROWS_PER_TILE, zfill, 0)

    off = s * ROWS_PER_TILE
    pltpu.sync_copy(zb, acc.at[pl.ds(off, ROWS_PER_TILE)])
    pltpu.sync_copy(zb, cnt.at[pl.ds(off, ROWS_PER_TILE)])
    plsc.subcore_barrier()

    pltpu.sync_copy(src_hbm.at[w], idx_buf)

    base = w * E_PER_TILE

    def stage_body(st, _):
        pltpu.sync_copy(
            ea_hbm.at[pl.ds(base + st * (STAGE_CHUNKS * CHUNK),
                            STAGE_CHUNKS * CHUNK)], ea_buf)
        for j in range(STAGE_CHUNKS):
            k = st * STAGE_CHUNKS + j
            pltpu.sync_copy(ea_buf.at[pl.ds(j * CHUNK, CHUNK)],
                            acc.at[idx_buf.at[k]], add=True)
            pltpu.sync_copy(ones_buf, cnt.at[idx_buf.at[k]], add=True)
        return 0
    lax.fori_loop(0, N_STAGES, stage_body, 0)

    plsc.subcore_barrier()

    pltpu.sync_copy(acc.at[pl.ds(off, ROWS_PER_TILE)], zb)
    pltpu.sync_copy(zb, sums_out.at[c, pl.ds(off, ROWS_PER_TILE)])
    pltpu.sync_copy(cnt.at[pl.ds(off, ROWS_PER_TILE)], zb)
    pltpu.sync_copy(zb, cnts_out.at[c, pl.ds(off, ROWS_PER_TILE)])


def _sc_scatter(ea_pad, src_pad):
    mesh = plsc.VectorSubcoreMesh(core_axis_name="c", subcore_axis_name="s")
    fn = pl.kernel(
        _sc_scatter_body,
        out_type=(
            jax.ShapeDtypeStruct((NC, N_NODES_P, 16), jnp.float32),
            jax.ShapeDtypeStruct((NC, N_NODES_P, 16), jnp.float32),
        ),
        mesh=mesh,
        compiler_params=pltpu.CompilerParams(use_tc_tiling_on_sc=False),
        scratch_types=[
            pltpu.VMEM((N_CHUNKS, CHUNK), jnp.int32),
            pltpu.VMEM((STAGE_CHUNKS * CHUNK, 16), jnp.float32),
            pltpu.VMEM((CHUNK, 16), jnp.float32),
            pltpu.VMEM((ROWS_PER_TILE, 16), jnp.float32),
            pltpu.VMEM_SHARED((N_NODES_P, 16), jnp.float32),
            pltpu.VMEM_SHARED((N_NODES_P, 16), jnp.float32),
        ],
    )
    return fn(ea_pad, src_pad)


def _mlp_body(x_ref, s_ref, c_ref, b_ref, u_ref, w1x_ref, w1e_ref, w1u_ref,
              b1_ref, w2_ref, b2_ref, o_ref):
    sums = s_ref[0] + s_ref[1]
    counts = c_ref[0, :, 0:1] + c_ref[1, :, 0:1]
    agg = sums / jnp.maximum(counts, 1.0)

    bvec = b_ref[...]
    gids = lax.broadcasted_iota(jnp.int32, (bvec.shape[0], 64), 1)
    onehot = (bvec == gids).astype(jnp.float32)

    uw = jnp.dot(u_ref[...], w1u_ref[...], preferred_element_type=jnp.float32)
    pre = (jnp.dot(x_ref[...], w1x_ref[...], preferred_element_type=jnp.float32)
           + jnp.dot(agg, w1e_ref[...], preferred_element_type=jnp.float32)
           + jnp.dot(onehot, uw, preferred_element_type=jnp.float32)
           + b1_ref[...])
    h = jnp.maximum(pre, 0.0)
    o_ref[...] = jnp.dot(h, w2_ref[...], preferred_element_type=jnp.float32) + b2_ref[...]


def _mlp(x, sums, cnts, batch2d, u, w1x, w1e, w1u, b1, w2, b2):
    n = x.shape[0]
    blk = 1000
    grid = n // blk
    return pl.pallas_call(
        _mlp_body,
        grid=(grid,),
        in_specs=[
            pl.BlockSpec((blk, 128), lambda i: (i, 0)),
            pl.BlockSpec((NC, blk, 16), lambda i: (0, i, 0)),
            pl.BlockSpec((NC, blk, 16), lambda i: (0, i, 0)),
            pl.BlockSpec((blk, 1), lambda i: (i, 0)),
            pl.BlockSpec((64, 128), lambda i: (0, 0)),
            pl.BlockSpec((128, 128), lambda i: (0, 0)),
            pl.BlockSpec((16, 128), lambda i: (0, 0)),
            pl.BlockSpec((128, 128), lambda i: (0, 0)),
            pl.BlockSpec((1, 128), lambda i: (0, 0)),
            pl.BlockSpec((128, 128), lambda i: (0, 0)),
            pl.BlockSpec((1, 128), lambda i: (0, 0)),
        ],
        out_specs=pl.BlockSpec((blk, 128), lambda i: (i, 0)),
        out_shape=jax.ShapeDtypeStruct((n, 128), jnp.float32),
    )(x, sums, cnts, batch2d, u, w1x, w1e, w1u, b1, w2, b2)


@jax.jit
def kernel(x, edge_index, edge_attr, u, batch, W1, b1, W2, b2):
    n_edges = edge_attr.shape[0]
    src = edge_index[0].astype(jnp.int32)
    src_pad = jnp.pad(src, (0, E_PAD - n_edges), constant_values=N_NODES_P - 1)
    src_pad = src_pad.reshape(NC * NS, N_CHUNKS, CHUNK)
    ea_pad = jnp.pad(edge_attr, ((0, E_PAD - n_edges), (0, 0)))

    sums, cnts = _sc_scatter(ea_pad, src_pad)

    w1x = W1[:128]
    w1e = W1[128:144]
    w1u = W1[144:]
    batch2d = batch.astype(jnp.int32).reshape(-1, 1)
    out = _mlp(x, sums, cnts, batch2d, u,
               w1x, w1e, w1u, b1.reshape(1, -1), W2, b2.reshape(1, -1))
    return out


# speedup vs baseline: 6.5622x; 1.5566x over previous
import jax
import jax.numpy as jnp
from jax import lax
from jax.experimental import pallas as pl
from jax.experimental.pallas import tpu as pltpu
from jax.experimental.pallas import tpu_sc as plsc

N_EDGES = 320000
N_NODES_P = 10240
DUMP_ROW = N_NODES_P - 1
E_PER_TILE = 10240
E_REAL = 10000
N_CHUNKS = 80
STAGE_CHUNKS = 8
N_STAGES = N_CHUNKS // STAGE_CHUNKS
STAGE_E = STAGE_CHUNKS * 128
CHUNK = 128
NC = 2
NS = 16
NW = NC * NS
ROWS_PER_TILE = N_NODES_P // NS


def _sc_scatter_body(ei_hbm, ea_hbm, sums_out, cnts_out,
                     raw_buf, idx_buf, ea_buf0, ea_buf1, ones_buf, zb,
                     acc, cnt, sem0, sem1, ssem):
    c = lax.axis_index("c")
    s = lax.axis_index("s")
    w = s * NC + c

    is_last = w == NW - 1
    base = jnp.where(is_last, N_EDGES - E_PER_TILE, w * E_REAL)
    lo = jnp.where(is_last, E_PER_TILE - E_REAL, 0)
    hi = jnp.where(is_last, E_PER_TILE, E_REAL)

    cp0 = pltpu.async_copy(ea_hbm.at[pl.ds(base, STAGE_E)], ea_buf0, sem0)
    cp1 = pltpu.async_copy(ea_hbm.at[pl.ds(base + STAGE_E, STAGE_E)],
                           ea_buf1, sem1)
    pltpu.sync_copy(ei_hbm.at[0, pl.ds(base, E_PER_TILE)], raw_buf)

    def fill(i, _):
        ones_buf[i, :] = jnp.full((16,), 1.0, dtype=jnp.float32)
        return 0
    lax.fori_loop(0, CHUNK, fill, 0)

    def zfill(i, _):
        zb[i, :] = jnp.zeros((16,), dtype=jnp.float32)
        return 0
    lax.fori_loop(0, ROWS_PER_TILE, zfill, 0)

    lane = lax.iota(jnp.int32, 16)

    def fix(i, _):
        for l in range(CHUNK // 16):
            slot = i * CHUNK + l * 16 + lane
            v = raw_buf[pl.ds(i * CHUNK + l * 16, 16)]
            ok = (slot >= lo) & (slot < hi)
            idx_buf[i, pl.ds(l * 16, 16)] = jnp.where(
                ok, v, jnp.full((16,), DUMP_ROW, dtype=jnp.int32))
        return 0
    lax.fori_loop(0, N_CHUNKS, fix, 0)

    off = s * ROWS_PER_TILE
    pltpu.sync_copy(zb, acc.at[pl.ds(off, ROWS_PER_TILE)])
    pltpu.sync_copy(zb, cnt.at[pl.ds(off, ROWS_PER_TILE)])
    plsc.subcore_barrier()

    bufs = (ea_buf0, ea_buf1)
    sems = (sem0, sem1)
    pend = {0: cp0, 1: cp1}
    for st in range(N_STAGES):
        buf = bufs[st % 2]
        pend.pop(st).wait()
        scats = []
        for j in range(STAGE_CHUNKS):
            k = st * STAGE_CHUNKS + j
            scats.append(pltpu.async_copy(
                buf.at[pl.ds(j * CHUNK, CHUNK)], acc.at[idx_buf.at[k]],
                ssem, add=True))
            scats.append(pltpu.async_copy(
                ones_buf, cnt.at[idx_buf.at[k]], ssem, add=True))
        for d in scats:
            d.wait()
        if st + 2 < N_STAGES:
            pend[st + 2] = pltpu.async_copy(
                ea_hbm.at[pl.ds(base + (st + 2) * STAGE_E, STAGE_E)],
                buf, sems[st % 2])

    plsc.subcore_barrier()

    pltpu.sync_copy(acc.at[pl.ds(off, ROWS_PER_TILE)], zb)
    pltpu.sync_copy(zb, sums_out.at[c, pl.ds(off, ROWS_PER_TILE)])
    pltpu.sync_copy(cnt.at[pl.ds(off, ROWS_PER_TILE)], zb)
    pltpu.sync_copy(zb, cnts_out.at[c, pl.ds(off, ROWS_PER_TILE)])


def _sc_scatter(edge_index, edge_attr):
    mesh = plsc.VectorSubcoreMesh(core_axis_name="c", subcore_axis_name="s")
    fn = pl.kernel(
        _sc_scatter_body,
        out_type=(
            jax.ShapeDtypeStruct((NC, N_NODES_P, 16), jnp.float32),
            jax.ShapeDtypeStruct((NC, N_NODES_P, 16), jnp.float32),
        ),
        mesh=mesh,
        compiler_params=pltpu.CompilerParams(use_tc_tiling_on_sc=False),
        scratch_types=[
            pltpu.VMEM((E_PER_TILE,), jnp.int32),
            pltpu.VMEM((N_CHUNKS, CHUNK), jnp.int32),
            pltpu.VMEM((STAGE_E, 16), jnp.float32),
            pltpu.VMEM((STAGE_E, 16), jnp.float32),
            pltpu.VMEM((CHUNK, 16), jnp.float32),
            pltpu.VMEM((ROWS_PER_TILE, 16), jnp.float32),
            pltpu.VMEM_SHARED((N_NODES_P, 16), jnp.float32),
            pltpu.VMEM_SHARED((N_NODES_P, 16), jnp.float32),
            pltpu.SemaphoreType.DMA,
            pltpu.SemaphoreType.DMA,
            pltpu.SemaphoreType.DMA,
        ],
    )
    return fn(edge_index, edge_attr)


def _mlp_body(x_ref, s_ref, c_ref, b_ref, u_ref, w1x_ref, w1e_ref, w1u_ref,
              b1_ref, w2_ref, b2_ref, o_ref):
    sums = s_ref[0] + s_ref[1]
    counts = c_ref[0, :, 0:1] + c_ref[1, :, 0:1]
    agg = sums / jnp.maximum(counts, 1.0)

    bvec = b_ref[...]
    gids = lax.broadcasted_iota(jnp.int32, (bvec.shape[0], 64), 1)
    onehot = (bvec == gids).astype(jnp.float32)

    uw = jnp.dot(u_ref[...], w1u_ref[...], preferred_element_type=jnp.float32)
    pre = (jnp.dot(x_ref[...], w1x_ref[...], preferred_element_type=jnp.float32)
           + jnp.dot(agg, w1e_ref[...], preferred_element_type=jnp.float32)
           + jnp.dot(onehot, uw, preferred_element_type=jnp.float32)
           + b1_ref[...])
    h = jnp.maximum(pre, 0.0)
    o_ref[...] = jnp.dot(h, w2_ref[...], preferred_element_type=jnp.float32) + b2_ref[...]


def _mlp(x, sums, cnts, batch2d, u, w1x, w1e, w1u, b1, w2, b2):
    n = x.shape[0]
    blk = 1000
    grid = n // blk
    return pl.pallas_call(
        _mlp_body,
        grid=(grid,),
        in_specs=[
            pl.BlockSpec((blk, 128), lambda i: (i, 0)),
            pl.BlockSpec((NC, blk, 16), lambda i: (0, i, 0)),
            pl.BlockSpec((NC, blk, 16), lambda i: (0, i, 0)),
            pl.BlockSpec((blk, 1), lambda i: (i, 0)),
            pl.BlockSpec((64, 128), lambda i: (0, 0)),
            pl.BlockSpec((128, 128), lambda i: (0, 0)),
            pl.BlockSpec((16, 128), lambda i: (0, 0)),
            pl.BlockSpec((128, 128), lambda i: (0, 0)),
            pl.BlockSpec((1, 128), lambda i: (0, 0)),
            pl.BlockSpec((128, 128), lambda i: (0, 0)),
            pl.BlockSpec((1, 128), lambda i: (0, 0)),
        ],
        out_specs=pl.BlockSpec((blk, 128), lambda i: (i, 0)),
        out_shape=jax.ShapeDtypeStruct((n, 128), jnp.float32),
    )(x, sums, cnts, batch2d, u, w1x, w1e, w1u, b1, w2, b2)


@jax.jit
def kernel(x, edge_index, edge_attr, u, batch, W1, b1, W2, b2):
    sums, cnts = _sc_scatter(edge_index.astype(jnp.int32), edge_attr)

    w1x = W1[:128]
    w1e = W1[128:144]
    w1u = W1[144:]
    batch2d = batch.astype(jnp.int32).reshape(-1, 1)
    out = _mlp(x, sums, cnts, batch2d, u,
               w1x, w1e, w1u, b1.reshape(1, -1), W2, b2.reshape(1, -1))
    return out
